# baseline (speedup 1.0000x reference)
import functools

import jax
import jax.numpy as jnp
from jax import lax
from jax.experimental import pallas as pl
from jax.experimental.pallas import tpu as pltpu
from jax.experimental.pallas import tpu_sc as plsc

_B = 4
_T = 4096
_D = 1024
_TD = _T * _D
_NW = 32
_WPW = _TD // _NW
_R = 16
_CW = _R * _D


def _sc_body(x_hbm, pe_hbm, out_hbm, pe_buf, xb):
    wid = lax.axis_index("s") * 2 + lax.axis_index("c")
    base = wid * _WPW
    for c in range(_WPW // _CW):
        off = base + c * _CW
        pltpu.sync_copy(pe_hbm.at[pl.ds(off, _CW)], pe_buf)
        for b in range(_B):
            pltpu.sync_copy(x_hbm.at[pl.ds(b * _TD + off, _CW)], xb.at[b])

        @plsc.parallel_loop(0, _CW // 16, 1, unroll=8)
        def _(k):
            s = pl.ds(k * 16, 16)
            pv = pe_buf[s]
            for b in range(_B):
                xb[b, s] = xb[b, s] + pv

        for b in range(_B):
            pltpu.sync_copy(xb.at[b], out_hbm.at[pl.ds(b * _TD + off, _CW)])


def _sc_add(x1, pe1):
    return pl.kernel(
        _sc_body,
        out_type=jax.ShapeDtypeStruct((_B * _TD,), jnp.float32),
        mesh=plsc.VectorSubcoreMesh(core_axis_name="c", subcore_axis_name="s"),
        scratch_types=[
            pltpu.VMEM((_CW,), jnp.float32),
            pltpu.VMEM((_B, _CW), jnp.float32),
        ],
    )(x1, pe1)


def kernel(x, pos_embed):
    B, T, D = x.shape
    out1 = _sc_add(x.reshape(-1), pos_embed.reshape(-1))
    return out1.reshape(B, T, D)


# speedup vs baseline: 4.6559x; 4.6559x over previous
import jax
import jax.numpy as jnp
from jax.experimental import pallas as pl


_BT = 512


def _add_block(x_ref, pe_ref, o_ref):
    o_ref[...] = x_ref[...] + pe_ref[...]


def kernel(x, pos_embed):
    B, T, D = x.shape
    grid = (T // _BT, B)
    return pl.pallas_call(
        _add_block,
        grid=grid,
        in_specs=[
            pl.BlockSpec((1, _BT, D), lambda i, b: (b, i, 0)),
            pl.BlockSpec((1, _BT, D), lambda i, b: (0, i, 0)),
        ],
        out_specs=pl.BlockSpec((1, _BT, D), lambda i, b: (b, i, 0)),
        out_shape=jax.ShapeDtypeStruct((B, T, D), x.dtype),
    )(x, pos_embed[None])


# speedup vs baseline: 5.2470x; 1.1270x over previous
import jax
import jax.numpy as jnp
from jax.experimental import pallas as pl


_BT = 512


def _add_block(x_ref, pe_ref, o_ref):
    o_ref[...] = x_ref[...] + pe_ref[...]


def kernel(x, pos_embed):
    B, T, D = x.shape
    grid = (T // _BT, B // 2)
    return pl.pallas_call(
        _add_block,
        grid=grid,
        in_specs=[
            pl.BlockSpec((2, _BT, D), lambda i, b: (b, i, 0)),
            pl.BlockSpec((1, _BT, D), lambda i, b: (0, i, 0)),
        ],
        out_specs=pl.BlockSpec((2, _BT, D), lambda i, b: (b, i, 0)),
        out_shape=jax.ShapeDtypeStruct((B, T, D), x.dtype),
    )(x, pos_embed[None])


# speedup vs baseline: 5.4508x; 1.0388x over previous
import jax
import jax.numpy as jnp
from jax.experimental import pallas as pl


_BT = 2048


def _add_block(x_ref, pe_ref, o_ref):
    o_ref[...] = x_ref[...] + pe_ref[...]


def kernel(x, pos_embed):
    B, T, D = x.shape
    grid = (T // _BT, B)
    return pl.pallas_call(
        _add_block,
        grid=grid,
        in_specs=[
            pl.BlockSpec((1, _BT, D), lambda i, b: (b, i, 0)),
            pl.BlockSpec((1, _BT, D), lambda i, b: (0, i, 0)),
        ],
        out_specs=pl.BlockSpec((1, _BT, D), lambda i, b: (b, i, 0)),
        out_shape=jax.ShapeDtypeStruct((B, T, D), x.dtype),
    )(x, pos_embed[None])
